# Initial kernel scaffold; baseline (speedup 1.0000x reference)
import jax
import jax.numpy as jnp
from jax.experimental import pallas as pl


def kernel(binary_list, weight_initial, weights, W_nn, b_nn, eval):
    raise NotImplementedError("write your pallas kernel here")


# speedup vs baseline: 4.3099x; 4.3099x over previous
import functools

import jax
import jax.numpy as jnp
from jax import lax
from jax.experimental import pallas as pl
from jax.experimental.pallas import tpu as pltpu
from jax.experimental.pallas import tpu_sc as plsc

L, B, D = 50, 1024, 784
NC, NS, LANES = 2, 16, 16
NW = NC * NS
BPW = B // NW
NG = BPW // LANES


def _tc_body(x_ref, w_ref, b_ref, dsym_ref, mdtv_ref):
    i = pl.program_id(0)
    x = x_ref[0]
    logits = jnp.dot(x, w_ref[...],
                     preferred_element_type=jnp.float32) + b_ref[0]
    l0 = logits[:, 0]
    l1 = logits[:, 1]
    dtv = jnp.maximum(l0, l1)
    dsym_ref[0, 0, :] = (l1 > l0).astype(jnp.int32)

    @pl.when(i == 0)
    def _():
        mdtv_ref[0, :] = dtv

    @pl.when(i > 0)
    def _():
        mdtv_ref[0, :] = jnp.minimum(mdtv_ref[0, :], dtv)


def _tc_stage(binary_list, W_nn, b_nn):
    return pl.pallas_call(
        _tc_body,
        grid=(L,),
        in_specs=[
            pl.BlockSpec((1, B, D), lambda i: (i, 0, 0)),
            pl.BlockSpec((D, 2), lambda i: (0, 0)),
            pl.BlockSpec((1, 2), lambda i: (0, 0)),
        ],
        out_specs=[
            pl.BlockSpec((1, 1, B), lambda i: (i, 0, 0)),
            pl.BlockSpec((1, B), lambda i: (0, 0)),
        ],
        out_shape=[
            jax.ShapeDtypeStruct((L, 1, B), jnp.int32),
            jax.ShapeDtypeStruct((1, B), jnp.float32),
        ],
    )(binary_list, W_nn, b_nn.reshape(1, 2))


def _sc_body(dsym_hbm, mdtv_hbm, pa_hbm, pb_hbm, pred_hbm, prev_hbm,
             dsym_v, mdtv_v, pa_v, pb_v, t_v, m_v, pred_v, prev_v):
    wid = lax.axis_index("s") * NC + lax.axis_index("c")
    blk = wid // 4
    sub = wid % 4
    base = blk * 128 + sub * BPW
    pltpu.sync_copy(pa_hbm, pa_v)
    pltpu.sync_copy(pb_hbm, pb_v)
    pltpu.sync_copy(dsym_hbm.at[:, pl.ds(pl.multiple_of(blk * 128, 128), 128)],
                    dsym_v)
    pltpu.sync_copy(mdtv_hbm.at[pl.ds(base, BPW)], mdtv_v)

    a = pa_v[...]
    b = pb_v[...]
    t_v[...] = 1.0 / (1.0 + jnp.exp(-jnp.abs(a - b)))
    m_v[...] = jnp.where(b > a,
                         jnp.full((LANES,), 1, jnp.int32),
                         jnp.full((LANES,), 0, jnp.int32))

    init_idx = jnp.full((LANES,), 4, jnp.int32)
    for g in range(NG):
        first_truth = plsc.load_gather(t_v, [init_idx])
        prev = plsc.load_gather(m_v, [init_idx])
        mn = first_truth
        for i in range(L):
            ds = dsym_v[i, pl.ds(sub * BPW + g * LANES, LANES)]
            prev = plsc.load_gather(m_v, [ds * 2 + prev])
            mn = jnp.minimum(mn, plsc.load_gather(t_v, [ds * 2 + prev]))
        mn = jnp.minimum(mn, mdtv_v[pl.ds(g * LANES, LANES)])
        pred_v[pl.ds(g * LANES, LANES)] = mn
        prev_v[pl.ds(g * LANES, LANES)] = prev

    pltpu.sync_copy(pred_v, pred_hbm.at[pl.ds(base, BPW)])
    pltpu.sync_copy(prev_v, prev_hbm.at[pl.ds(base, BPW)])


@functools.cache
def _sc_stage():
    return pl.kernel(
        _sc_body,
        out_type=(jax.ShapeDtypeStruct((B,), jnp.float32),
                  jax.ShapeDtypeStruct((B,), jnp.int32)),
        mesh=plsc.VectorSubcoreMesh(core_axis_name="c", subcore_axis_name="s"),
        compiler_params=pltpu.CompilerParams(needs_layout_passes=False),
        scratch_types=[
            pltpu.VMEM((L, 128), jnp.int32),
            pltpu.VMEM((BPW,), jnp.float32),
            pltpu.VMEM((LANES,), jnp.float32),
            pltpu.VMEM((LANES,), jnp.float32),
            pltpu.VMEM((LANES,), jnp.float32),
            pltpu.VMEM((LANES,), jnp.int32),
            pltpu.VMEM((BPW,), jnp.float32),
            pltpu.VMEM((BPW,), jnp.int32),
        ],
    )


def kernel(binary_list, weight_initial, weights, W_nn, b_nn, eval):
    dsym, mdtv = _tc_stage(binary_list, W_nn, b_nn)
    wflat = weights.reshape(4, 2)
    pad = jnp.zeros((LANES - 5,), jnp.float32)
    pa = jnp.concatenate([wflat[:, 0], weight_initial[:, 0], pad])
    pb = jnp.concatenate([wflat[:, 1], weight_initial[:, 1], pad])
    pred, prev = _sc_stage()(dsym.reshape(L, B), mdtv.reshape(B), pa, pb)
    return pred, prev.reshape(B, 1, 1)


# speedup vs baseline: 4.4997x; 1.0440x over previous
import functools

import jax
import jax.numpy as jnp
from jax import lax
from jax.experimental import pallas as pl
from jax.experimental.pallas import tpu as pltpu
from jax.experimental.pallas import tpu_sc as plsc

L, B, D = 50, 1024, 784
NC, NS, LANES = 2, 16, 16
NW = NC * NS
BPW = B // NW
NG = BPW // LANES
B_TC = 256
B_SC = B - B_TC
RPW = B_SC // NW
KC = D // LANES
KU = 7


def _tc_body(x_ref, w_ref, b_ref, dsym_ref, mdtv_ref):
    i = pl.program_id(0)
    x = x_ref[0]
    logits = jnp.dot(x, w_ref[...],
                     preferred_element_type=jnp.float32) + b_ref[0]
    l0 = logits[:, 0]
    l1 = logits[:, 1]
    dtv = jnp.maximum(l0, l1)
    dsym_ref[0, 0, :] = (l1 > l0).astype(jnp.int32)

    @pl.when(i == 0)
    def _():
        mdtv_ref[0, :] = dtv

    @pl.when(i > 0)
    def _():
        mdtv_ref[0, :] = jnp.minimum(mdtv_ref[0, :], dtv)


def _tc_stage(binary_list, W_nn, b_nn):
    return pl.pallas_call(
        _tc_body,
        grid=(L,),
        in_specs=[
            pl.BlockSpec((1, B_TC, D), lambda i: (i, 0, 0)),
            pl.BlockSpec((D, 2), lambda i: (0, 0)),
            pl.BlockSpec((1, 2), lambda i: (0, 0)),
        ],
        out_specs=[
            pl.BlockSpec((1, 1, B_TC), lambda i: (i, 0, 0)),
            pl.BlockSpec((1, B_TC), lambda i: (0, 0)),
        ],
        out_shape=[
            jax.ShapeDtypeStruct((L, 1, B_TC), jnp.int32),
            jax.ShapeDtypeStruct((1, B_TC), jnp.float32),
        ],
    )(binary_list, W_nn, b_nn.reshape(1, 2))


def _mv_compute(xb, wv_v, bv_v, ds_v, md_v, i):
    b0 = bv_v[0, :]
    b1 = bv_v[1, :]
    lanes = lax.iota(jnp.int32, LANES)
    one = jnp.full((LANES,), 1, jnp.int32)
    zero = jnp.full((LANES,), 0, jnp.int32)
    for grp in range(2):
        nrb = 4 if grp == 0 else 2
        s0v = jnp.full((LANES,), jnp.float32(jnp.inf))
        s1v = jnp.full((LANES,), jnp.float32(jnp.inf))
        for rb in range(nrb):
            r0 = grp * LANES + rb * 4

            def kbody(t, accs, _r0=r0):
                a = list(accs)
                for c in range(KU):
                    off = t * (KU * LANES) + c * LANES
                    w0 = wv_v[0, pl.ds(off, LANES)]
                    w1 = wv_v[1, pl.ds(off, LANES)]
                    for q in range(4):
                        xv = xb[_r0 + q, pl.ds(off, LANES)]
                        a[q] = a[q] + xv * w0
                        a[4 + q] = a[4 + q] + xv * w1
                return tuple(a)

            accs = lax.fori_loop(0, KC // KU, kbody,
                                 (b0, b0, b0, b0, b1, b1, b1, b1))
            for q in range(4):
                msk = lanes == (rb * 4 + q)
                s0v = jnp.where(msk, jnp.full((LANES,), jnp.sum(accs[q])), s0v)
                s1v = jnp.where(msk, jnp.full((LANES,), jnp.sum(accs[4 + q])),
                                s1v)
        dsv = jnp.where(s1v > s0v, one, zero)
        dtv = jnp.maximum(s0v, s1v)
        plsc.store_scatter(ds_v, [lanes + grp * LANES,
                                  jnp.full((LANES,), i, jnp.int32)], dsv)
        sl = pl.ds(grp * LANES, LANES)
        md_v[sl] = jnp.minimum(md_v[sl], dtv)


def _sc_mv_body(xs_hbm, wv_hbm, bv_hbm, dsym_hbm, mdtv_hbm,
                xb0, xb1, wv_v, bv_v, ds_v, md_v, sem0, sem1):
    wid = lax.axis_index("s") * NC + lax.axis_index("c")
    row0 = pl.multiple_of(B_TC + wid * RPW, 8)
    pltpu.sync_copy(wv_hbm, wv_v)
    pltpu.sync_copy(bv_hbm, bv_v)
    inf16 = jnp.full((LANES,), jnp.float32(jnp.inf))
    md_v[pl.ds(0, LANES)] = inf16
    md_v[pl.ds(LANES, LANES)] = inf16

    def cp(step, buf, sem):
        return pltpu.make_async_copy(
            xs_hbm.at[step, pl.ds(row0, RPW), :], buf, sem)

    cp(0, xb0, sem0).start()
    cp(1, xb1, sem1).start()

    def pair_body(j, _):
        i0 = 2 * j
        i1 = 2 * j + 1
        cp(0, xb0, sem0).wait()
        _mv_compute(xb0, wv_v, bv_v, ds_v, md_v, i0)
        cp(jnp.minimum(i0 + 2, L - 1), xb0, sem0).start()
        cp(0, xb1, sem1).wait()
        _mv_compute(xb1, wv_v, bv_v, ds_v, md_v, i1)
        cp(jnp.minimum(i1 + 2, L - 1), xb1, sem1).start()
        return 0

    lax.fori_loop(0, L // 2, pair_body, 0)
    cp(0, xb0, sem0).wait()
    cp(0, xb1, sem1).wait()

    obase = pl.multiple_of(wid * RPW, 8)
    pltpu.sync_copy(ds_v.at[pl.ds(0, RPW), :],
                    dsym_hbm.at[pl.ds(obase, RPW), :])
    pltpu.sync_copy(md_v.at[pl.ds(0, RPW)],
                    mdtv_hbm.at[pl.ds(obase, RPW)])


@functools.cache
def _sc_mv_stage():
    return pl.kernel(
        _sc_mv_body,
        out_type=(jax.ShapeDtypeStruct((B_SC, L), jnp.int32),
                  jax.ShapeDtypeStruct((B_SC,), jnp.float32)),
        mesh=plsc.VectorSubcoreMesh(core_axis_name="c", subcore_axis_name="s"),
        compiler_params=pltpu.CompilerParams(needs_layout_passes=False),
        scratch_types=[
            pltpu.VMEM((RPW, D), jnp.float32),
            pltpu.VMEM((RPW, D), jnp.float32),
            pltpu.VMEM((2, D), jnp.float32),
            pltpu.VMEM((2, LANES), jnp.float32),
            pltpu.VMEM((2 * LANES, L), jnp.int32),
            pltpu.VMEM((2 * LANES,), jnp.float32),
            pltpu.SemaphoreType.DMA,
            pltpu.SemaphoreType.DMA,
        ],
    )


def _sc_auto_body(dsym_tc, dsym_sc, mdtv_tc, mdtv_sc, pa_hbm, pb_hbm,
                  pred_hbm, prev_hbm,
                  dtc_v, dsc_v, mtc_v, msc_v, pa_v, pb_v, t_v, m_v,
                  pred_v, prev_v):
    wid = lax.axis_index("s") * NC + lax.axis_index("c")
    base = pl.multiple_of(wid * BPW, 32)
    blk = wid // 4
    sub = wid % 4
    wtc = jnp.where(wid < B_TC // BPW, jnp.int32(1), jnp.int32(0))
    wtcf = wtc.astype(jnp.float32)
    tc_blk = jnp.minimum(blk, B_TC // 128 - 1)
    sc_off = pl.multiple_of(jnp.clip(base - B_TC, 0, B_SC - BPW), 32)
    mtc_off = pl.multiple_of(jnp.minimum(base, B_TC - BPW), 32)

    pltpu.sync_copy(pa_hbm, pa_v)
    pltpu.sync_copy(pb_hbm, pb_v)
    pltpu.sync_copy(
        dsym_tc.at[:, pl.ds(pl.multiple_of(tc_blk * 128, 128), 128)], dtc_v)
    pltpu.sync_copy(dsym_sc.at[pl.ds(sc_off, BPW), :], dsc_v)
    pltpu.sync_copy(mdtv_tc.at[pl.ds(mtc_off, BPW)], mtc_v)
    pltpu.sync_copy(mdtv_sc.at[pl.ds(sc_off, BPW)], msc_v)

    a = pa_v[...]
    b = pb_v[...]
    t_v[...] = 1.0 / (1.0 + jnp.exp(-jnp.abs(a - b)))
    m_v[...] = jnp.where(b > a,
                         jnp.full((LANES,), 1, jnp.int32),
                         jnp.full((LANES,), 0, jnp.int32))

    init_idx = jnp.full((LANES,), 4, jnp.int32)
    rows16 = lax.iota(jnp.int32, LANES)
    for g in range(NG):
        first_truth = plsc.load_gather(t_v, [init_idx])
        prev = plsc.load_gather(m_v, [init_idx])
        mn = first_truth
        for i in range(L):
            ds_tc = dtc_v[i, pl.ds(sub * BPW + g * LANES, LANES)]
            ds_sc = plsc.load_gather(
                dsc_v, [rows16 + g * LANES, jnp.full((LANES,), i, jnp.int32)])
            ds = ds_tc * wtc + ds_sc * (1 - wtc)
            prev = plsc.load_gather(m_v, [ds * 2 + prev])
            mn = jnp.minimum(mn, plsc.load_gather(t_v, [ds * 2 + prev]))
        md_tc = mtc_v[pl.ds(g * LANES, LANES)]
        md_sc = msc_v[pl.ds(g * LANES, LANES)]
        mn = jnp.minimum(mn, md_tc * wtcf + md_sc * (1.0 - wtcf))
        pred_v[pl.ds(g * LANES, LANES)] = mn
        prev_v[pl.ds(g * LANES, LANES)] = prev

    pltpu.sync_copy(pred_v, pred_hbm.at[pl.ds(base, BPW)])
    pltpu.sync_copy(prev_v, prev_hbm.at[pl.ds(base, BPW)])


@functools.cache
def _sc_auto_stage():
    return pl.kernel(
        _sc_auto_body,
        out_type=(jax.ShapeDtypeStruct((B,), jnp.float32),
                  jax.ShapeDtypeStruct((B,), jnp.int32)),
        mesh=plsc.VectorSubcoreMesh(core_axis_name="c", subcore_axis_name="s"),
        compiler_params=pltpu.CompilerParams(needs_layout_passes=False),
        scratch_types=[
            pltpu.VMEM((L, 128), jnp.int32),
            pltpu.VMEM((BPW, L), jnp.int32),
            pltpu.VMEM((BPW,), jnp.float32),
            pltpu.VMEM((BPW,), jnp.float32),
            pltpu.VMEM((LANES,), jnp.float32),
            pltpu.VMEM((LANES,), jnp.float32),
            pltpu.VMEM((LANES,), jnp.float32),
            pltpu.VMEM((LANES,), jnp.int32),
            pltpu.VMEM((BPW,), jnp.float32),
            pltpu.VMEM((BPW,), jnp.int32),
        ],
    )


def kernel(binary_list, weight_initial, weights, W_nn, b_nn, eval):
    dsym_tc, mdtv_tc = _tc_stage(binary_list, W_nn, b_nn)
    wv = W_nn.T
    bv = jnp.stack([jnp.full((LANES,), b_nn[0] / LANES),
                    jnp.full((LANES,), b_nn[1] / LANES)])
    dsym_sc, mdtv_sc = _sc_mv_stage()(binary_list, wv, bv)
    wflat = weights.reshape(4, 2)
    pad = jnp.zeros((LANES - 5,), jnp.float32)
    pa = jnp.concatenate([wflat[:, 0], weight_initial[:, 0], pad])
    pb = jnp.concatenate([wflat[:, 1], weight_initial[:, 1], pad])
    pred, prev = _sc_auto_stage()(
        dsym_tc.reshape(L, B_TC), dsym_sc, mdtv_tc.reshape(B_TC), mdtv_sc,
        pa, pb)
    return pred, prev.reshape(B, 1, 1)


# speedup vs baseline: 4.4998x; 1.0000x over previous
import functools

import jax
import jax.numpy as jnp
from jax import lax
from jax.experimental import pallas as pl
from jax.experimental.pallas import tpu as pltpu
from jax.experimental.pallas import tpu_sc as plsc

L, B, D = 50, 1024, 784
NC, NS, LANES = 2, 16, 16
NW = NC * NS
BPW = B // NW
NG = BPW // LANES
B_TC = 256
B_SC = B - B_TC
RPW = B_SC // NW
KC = D // LANES
KU = 7


def _tc_body(x_ref, w_ref, b_ref, dsym_ref, mdtv_ref):
    i = pl.program_id(0)
    x = x_ref[0]
    logits = jnp.dot(x, w_ref[...],
                     preferred_element_type=jnp.float32) + b_ref[0]
    l0 = logits[:, 0]
    l1 = logits[:, 1]
    dtv = jnp.maximum(l0, l1)
    dsym_ref[0, 0, :] = (l1 > l0).astype(jnp.int32)

    @pl.when(i == 0)
    def _():
        mdtv_ref[0, :] = dtv

    @pl.when(i > 0)
    def _():
        mdtv_ref[0, :] = jnp.minimum(mdtv_ref[0, :], dtv)


def _tc_stage(binary_list, W_nn, b_nn):
    return pl.pallas_call(
        _tc_body,
        grid=(L,),
        in_specs=[
            pl.BlockSpec((1, B_TC, D), lambda i: (i, 0, 0)),
            pl.BlockSpec((D, 2), lambda i: (0, 0)),
            pl.BlockSpec((1, 2), lambda i: (0, 0)),
        ],
        out_specs=[
            pl.BlockSpec((1, 1, B_TC), lambda i: (i, 0, 0)),
            pl.BlockSpec((1, B_TC), lambda i: (0, 0)),
        ],
        out_shape=[
            jax.ShapeDtypeStruct((L, 1, B_TC), jnp.int32),
            jax.ShapeDtypeStruct((1, B_TC), jnp.float32),
        ],
    )(binary_list, W_nn, b_nn.reshape(1, 2))


def _mv_compute(xb, wv_v, bv_v, ds_v, md_v, i):
    b0 = bv_v[0, :]
    b1 = bv_v[1, :]
    lanes = lax.iota(jnp.int32, LANES)
    one = jnp.full((LANES,), 1, jnp.int32)
    zero = jnp.full((LANES,), 0, jnp.int32)
    for grp in range(2):
        nrb = 4 if grp == 0 else 2
        s0v = jnp.full((LANES,), jnp.float32(jnp.inf))
        s1v = jnp.full((LANES,), jnp.float32(jnp.inf))
        for rb in range(nrb):
            r0 = grp * LANES + rb * 4

            def kbody(t, accs, _r0=r0):
                a = list(accs)
                for c in range(KU):
                    off = t * (KU * LANES) + c * LANES
                    w0 = wv_v[0, pl.ds(off, LANES)]
                    w1 = wv_v[1, pl.ds(off, LANES)]
                    for q in range(4):
                        xv = xb[_r0 + q, pl.ds(off, LANES)]
                        a[q] = a[q] + xv * w0
                        a[4 + q] = a[4 + q] + xv * w1
                return tuple(a)

            accs = lax.fori_loop(0, KC // KU, kbody,
                                 (b0, b0, b0, b0, b1, b1, b1, b1))
            for q in range(4):
                msk = lanes == (rb * 4 + q)
                s0v = jnp.where(msk, jnp.full((LANES,), jnp.sum(accs[q])), s0v)
                s1v = jnp.where(msk, jnp.full((LANES,), jnp.sum(accs[4 + q])),
                                s1v)
        dsv = jnp.where(s1v > s0v, one, zero)
        dtv = jnp.maximum(s0v, s1v)
        plsc.store_scatter(ds_v, [lanes + grp * LANES,
                                  jnp.full((LANES,), i, jnp.int32)], dsv)
        sl = pl.ds(grp * LANES, LANES)
        md_v[sl] = jnp.minimum(md_v[sl], dtv)


def _sc_mv_body(xs_hbm, wv_hbm, bv_hbm, dsym_hbm, mdtv_hbm,
                xb0, xb1, wv_v, bv_v, ds_v, md_v, sem0, sem1):
    wid = lax.axis_index("s") * NC + lax.axis_index("c")
    row0 = pl.multiple_of(B_TC + wid * RPW, 8)
    pltpu.sync_copy(wv_hbm, wv_v)
    pltpu.sync_copy(bv_hbm, bv_v)
    inf16 = jnp.full((LANES,), jnp.float32(jnp.inf))
    md_v[pl.ds(0, LANES)] = inf16
    md_v[pl.ds(LANES, LANES)] = inf16

    def cp(step, buf, sem):
        return pltpu.make_async_copy(
            xs_hbm.at[step, pl.ds(row0, RPW), :], buf, sem)

    cp(0, xb0, sem0).start()
    cp(1, xb1, sem1).start()

    def pair_body(j, _):
        i0 = 2 * j
        i1 = 2 * j + 1
        cp(0, xb0, sem0).wait()
        _mv_compute(xb0, wv_v, bv_v, ds_v, md_v, i0)
        cp(jnp.minimum(i0 + 2, L - 1), xb0, sem0).start()
        cp(0, xb1, sem1).wait()
        _mv_compute(xb1, wv_v, bv_v, ds_v, md_v, i1)
        cp(jnp.minimum(i1 + 2, L - 1), xb1, sem1).start()
        return 0

    lax.fori_loop(0, L // 2, pair_body, 0)
    cp(0, xb0, sem0).wait()
    cp(0, xb1, sem1).wait()

    obase = pl.multiple_of(wid * RPW, 8)
    pltpu.sync_copy(ds_v.at[pl.ds(0, RPW), :],
                    dsym_hbm.at[pl.ds(obase, RPW), :])
    pltpu.sync_copy(md_v.at[pl.ds(0, RPW)],
                    mdtv_hbm.at[pl.ds(obase, RPW)])


@functools.cache
def _sc_mv_stage():
    return pl.kernel(
        _sc_mv_body,
        out_type=(jax.ShapeDtypeStruct((B_SC, L), jnp.int32),
                  jax.ShapeDtypeStruct((B_SC,), jnp.float32)),
        mesh=plsc.VectorSubcoreMesh(core_axis_name="c", subcore_axis_name="s"),
        compiler_params=pltpu.CompilerParams(needs_layout_passes=False,
                                             use_tc_tiling_on_sc=True),
        scratch_types=[
            pltpu.VMEM((RPW, D), jnp.float32),
            pltpu.VMEM((RPW, D), jnp.float32),
            pltpu.VMEM((2, D), jnp.float32),
            pltpu.VMEM((2, LANES), jnp.float32),
            pltpu.VMEM((2 * LANES, L), jnp.int32),
            pltpu.VMEM((2 * LANES,), jnp.float32),
            pltpu.SemaphoreType.DMA,
            pltpu.SemaphoreType.DMA,
        ],
    )


def _sc_auto_body(dsym_tc, dsym_sc, mdtv_tc, mdtv_sc, pa_hbm, pb_hbm,
                  pred_hbm, prev_hbm,
                  dtc_v, dsc_v, mtc_v, msc_v, pa_v, pb_v, t_v, m_v,
                  pred_v, prev_v):
    wid = lax.axis_index("s") * NC + lax.axis_index("c")
    base = pl.multiple_of(wid * BPW, 32)
    blk = wid // 4
    sub = wid % 4
    wtc = jnp.where(wid < B_TC // BPW, jnp.int32(1), jnp.int32(0))
    wtcf = wtc.astype(jnp.float32)
    tc_blk = jnp.minimum(blk, B_TC // 128 - 1)
    sc_off = pl.multiple_of(jnp.clip(base - B_TC, 0, B_SC - BPW), 32)
    mtc_off = pl.multiple_of(jnp.minimum(base, B_TC - BPW), 32)

    pltpu.sync_copy(pa_hbm, pa_v)
    pltpu.sync_copy(pb_hbm, pb_v)
    pltpu.sync_copy(
        dsym_tc.at[:, pl.ds(pl.multiple_of(tc_blk * 128, 128), 128)], dtc_v)
    pltpu.sync_copy(dsym_sc.at[pl.ds(sc_off, BPW), :], dsc_v)
    pltpu.sync_copy(mdtv_tc.at[pl.ds(mtc_off, BPW)], mtc_v)
    pltpu.sync_copy(mdtv_sc.at[pl.ds(sc_off, BPW)], msc_v)

    a = pa_v[...]
    b = pb_v[...]
    t_v[...] = 1.0 / (1.0 + jnp.exp(-jnp.abs(a - b)))
    m_v[...] = jnp.where(b > a,
                         jnp.full((LANES,), 1, jnp.int32),
                         jnp.full((LANES,), 0, jnp.int32))

    init_idx = jnp.full((LANES,), 4, jnp.int32)
    rows16 = lax.iota(jnp.int32, LANES)
    for g in range(NG):
        first_truth = plsc.load_gather(t_v, [init_idx])
        prev = plsc.load_gather(m_v, [init_idx])
        mn = first_truth
        for i in range(L):
            ds_tc = dtc_v[i, pl.ds(sub * BPW + g * LANES, LANES)]
            ds_sc = plsc.load_gather(
                dsc_v, [rows16 + g * LANES, jnp.full((LANES,), i, jnp.int32)])
            ds = ds_tc * wtc + ds_sc * (1 - wtc)
            prev = plsc.load_gather(m_v, [ds * 2 + prev])
            mn = jnp.minimum(mn, plsc.load_gather(t_v, [ds * 2 + prev]))
        md_tc = mtc_v[pl.ds(g * LANES, LANES)]
        md_sc = msc_v[pl.ds(g * LANES, LANES)]
        mn = jnp.minimum(mn, md_tc * wtcf + md_sc * (1.0 - wtcf))
        pred_v[pl.ds(g * LANES, LANES)] = mn
        prev_v[pl.ds(g * LANES, LANES)] = prev

    pltpu.sync_copy(pred_v, pred_hbm.at[pl.ds(base, BPW)])
    pltpu.sync_copy(prev_v, prev_hbm.at[pl.ds(base, BPW)])


@functools.cache
def _sc_auto_stage():
    return pl.kernel(
        _sc_auto_body,
        out_type=(jax.ShapeDtypeStruct((B,), jnp.float32),
                  jax.ShapeDtypeStruct((B,), jnp.int32)),
        mesh=plsc.VectorSubcoreMesh(core_axis_name="c", subcore_axis_name="s"),
        compiler_params=pltpu.CompilerParams(needs_layout_passes=False),
        scratch_types=[
            pltpu.VMEM((L, 128), jnp.int32),
            pltpu.VMEM((BPW, L), jnp.int32),
            pltpu.VMEM((BPW,), jnp.float32),
            pltpu.VMEM((BPW,), jnp.float32),
            pltpu.VMEM((LANES,), jnp.float32),
            pltpu.VMEM((LANES,), jnp.float32),
            pltpu.VMEM((LANES,), jnp.float32),
            pltpu.VMEM((LANES,), jnp.int32),
            pltpu.VMEM((BPW,), jnp.float32),
            pltpu.VMEM((BPW,), jnp.int32),
        ],
    )


def kernel(binary_list, weight_initial, weights, W_nn, b_nn, eval):
    dsym_tc, mdtv_tc = _tc_stage(binary_list, W_nn, b_nn)
    wv = W_nn.T
    bv = jnp.stack([jnp.full((LANES,), b_nn[0] / LANES),
                    jnp.full((LANES,), b_nn[1] / LANES)])
    dsym_sc, mdtv_sc = _sc_mv_stage()(binary_list, wv, bv)
    wflat = weights.reshape(4, 2)
    pad = jnp.zeros((LANES - 5,), jnp.float32)
    pa = jnp.concatenate([wflat[:, 0], weight_initial[:, 0], pad])
    pb = jnp.concatenate([wflat[:, 1], weight_initial[:, 1], pad])
    pred, prev = _sc_auto_stage()(
        dsym_tc.reshape(L, B_TC), dsym_sc, mdtv_tc.reshape(B_TC), mdtv_sc,
        pa, pb)
    return pred, prev.reshape(B, 1, 1)


# speedup vs baseline: 13.5681x; 3.0153x over previous
import functools

import jax
import jax.numpy as jnp
from jax import lax
from jax.experimental import pallas as pl
from jax.experimental.pallas import tpu as pltpu
from jax.experimental.pallas import tpu_sc as plsc

L, B, D = 50, 1024, 784
NC, NS, LANES = 2, 16, 16
NW = NC * NS
BPW = B // NW
NG = BPW // LANES


def _tc_body(x_ref, wt_ref, b_ref, dsym_ref, mdtv_ref):
    i = pl.program_id(0)
    x = x_ref[0]
    logits = jnp.dot(wt_ref[...], x,
                     preferred_element_type=jnp.float32) + b_ref[...]
    l0 = logits[0]
    l1 = logits[1]
    dtv = jnp.maximum(l0, l1)
    dsym_ref[0, 0, :] = (l1 > l0).astype(jnp.int32)

    @pl.when(i == 0)
    def _():
        mdtv_ref[0, :] = dtv

    @pl.when(i > 0)
    def _():
        mdtv_ref[0, :] = jnp.minimum(mdtv_ref[0, :], dtv)


def _tc_stage(xt, wt, b_nn):
    return pl.pallas_call(
        _tc_body,
        grid=(L,),
        in_specs=[
            pl.BlockSpec((1, D, B), lambda i: (i, 0, 0)),
            pl.BlockSpec((2, D), lambda i: (0, 0)),
            pl.BlockSpec((2, 1), lambda i: (0, 0)),
        ],
        out_specs=[
            pl.BlockSpec((1, 1, B), lambda i: (i, 0, 0)),
            pl.BlockSpec((1, B), lambda i: (0, 0)),
        ],
        out_shape=[
            jax.ShapeDtypeStruct((L, 1, B), jnp.int32),
            jax.ShapeDtypeStruct((1, B), jnp.float32),
        ],
    )(xt, wt, b_nn.reshape(2, 1))


def _sc_body(dsym_hbm, mdtv_hbm, pa_hbm, pb_hbm, pred_hbm, prev_hbm,
             dsym_v, mdtv_v, pa_v, pb_v, t_v, m_v, pred_v, prev_v):
    wid = lax.axis_index("s") * NC + lax.axis_index("c")
    blk = wid // 4
    sub = wid % 4
    base = blk * 128 + sub * BPW
    pltpu.sync_copy(pa_hbm, pa_v)
    pltpu.sync_copy(pb_hbm, pb_v)
    pltpu.sync_copy(dsym_hbm.at[:, pl.ds(pl.multiple_of(blk * 128, 128), 128)],
                    dsym_v)
    pltpu.sync_copy(mdtv_hbm.at[pl.ds(base, BPW)], mdtv_v)

    a = pa_v[...]
    b = pb_v[...]
    t_v[...] = 1.0 / (1.0 + jnp.exp(-jnp.abs(a - b)))
    m_v[...] = jnp.where(b > a,
                         jnp.full((LANES,), 1, jnp.int32),
                         jnp.full((LANES,), 0, jnp.int32))

    init_idx = jnp.full((LANES,), 4, jnp.int32)
    for g in range(NG):
        first_truth = plsc.load_gather(t_v, [init_idx])
        prev = plsc.load_gather(m_v, [init_idx])
        mn = first_truth
        for i in range(L):
            ds = dsym_v[i, pl.ds(sub * BPW + g * LANES, LANES)]
            prev = plsc.load_gather(m_v, [ds * 2 + prev])
            mn = jnp.minimum(mn, plsc.load_gather(t_v, [ds * 2 + prev]))
        mn = jnp.minimum(mn, mdtv_v[pl.ds(g * LANES, LANES)])
        pred_v[pl.ds(g * LANES, LANES)] = mn
        prev_v[pl.ds(g * LANES, LANES)] = prev

    pltpu.sync_copy(pred_v, pred_hbm.at[pl.ds(base, BPW)])
    pltpu.sync_copy(prev_v, prev_hbm.at[pl.ds(base, BPW)])


@functools.cache
def _sc_stage():
    return pl.kernel(
        _sc_body,
        out_type=(jax.ShapeDtypeStruct((B,), jnp.float32),
                  jax.ShapeDtypeStruct((B,), jnp.int32)),
        mesh=plsc.VectorSubcoreMesh(core_axis_name="c", subcore_axis_name="s"),
        compiler_params=pltpu.CompilerParams(needs_layout_passes=False),
        scratch_types=[
            pltpu.VMEM((L, 128), jnp.int32),
            pltpu.VMEM((BPW,), jnp.float32),
            pltpu.VMEM((LANES,), jnp.float32),
            pltpu.VMEM((LANES,), jnp.float32),
            pltpu.VMEM((LANES,), jnp.float32),
            pltpu.VMEM((LANES,), jnp.int32),
            pltpu.VMEM((BPW,), jnp.float32),
            pltpu.VMEM((BPW,), jnp.int32),
        ],
    )


def kernel(binary_list, weight_initial, weights, W_nn, b_nn, eval):
    xt = jnp.transpose(binary_list, (0, 2, 1))
    dsym, mdtv = _tc_stage(xt, W_nn.T, b_nn)
    wflat = weights.reshape(4, 2)
    pad = jnp.zeros((LANES - 5,), jnp.float32)
    pa = jnp.concatenate([wflat[:, 0], weight_initial[:, 0], pad])
    pb = jnp.concatenate([wflat[:, 1], weight_initial[:, 1], pad])
    pred, prev = _sc_stage()(dsym.reshape(L, B), mdtv.reshape(B), pa, pb)
    return pred, prev.reshape(B, 1, 1)


# speedup vs baseline: 13.6566x; 1.0065x over previous
import functools

import jax
import jax.numpy as jnp
from jax import lax
from jax.experimental import pallas as pl
from jax.experimental.pallas import tpu as pltpu
from jax.experimental.pallas import tpu_sc as plsc

L, B, D = 50, 1024, 784
NC, NS, LANES = 2, 16, 16
NW = NC * NS
BPW = B // NW
NG = BPW // LANES


def _tc_body(x_ref, wt_ref, b_ref, dsym_ref, mdtv_ref):
    i = pl.program_id(0)
    x = x_ref[0]
    logits = jnp.dot(wt_ref[...], x,
                     preferred_element_type=jnp.float32) + b_ref[...]
    l0 = logits[0]
    l1 = logits[1]
    dtv = jnp.maximum(l0, l1)
    dsym_ref[0, 0, :] = (l1 > l0).astype(jnp.int32)

    @pl.when(i == 0)
    def _():
        mdtv_ref[0, :] = dtv

    @pl.when(i > 0)
    def _():
        mdtv_ref[0, :] = jnp.minimum(mdtv_ref[0, :], dtv)


def _tc_stage(xt, wt, b_nn):
    return pl.pallas_call(
        _tc_body,
        grid=(L,),
        in_specs=[
            pl.BlockSpec((1, D, B), lambda i: (i, 0, 0)),
            pl.BlockSpec((2, D), lambda i: (0, 0)),
            pl.BlockSpec((2, 1), lambda i: (0, 0)),
        ],
        out_specs=[
            pl.BlockSpec((1, 1, B), lambda i: (i, 0, 0)),
            pl.BlockSpec((1, B), lambda i: (0, 0)),
        ],
        out_shape=[
            jax.ShapeDtypeStruct((L, 1, B), jnp.int32),
            jax.ShapeDtypeStruct((1, B), jnp.float32),
        ],
    )(xt, wt, b_nn.reshape(2, 1))


def _sc_body(dsym_hbm, mdtv_hbm, pa_hbm, pb_hbm, pred_hbm, prev_hbm,
             dsym_v, mdtv_v, pa_v, pb_v, t_v, m_v, pred_v, prev_v):
    wid = lax.axis_index("s") * NC + lax.axis_index("c")
    blk = wid // 4
    sub = wid % 4
    base = blk * 128 + sub * BPW
    pltpu.sync_copy(pa_hbm, pa_v)
    pltpu.sync_copy(pb_hbm, pb_v)
    pltpu.sync_copy(
        dsym_hbm.at[:, 0, pl.ds(pl.multiple_of(blk * 128, 128), 128)], dsym_v)
    pltpu.sync_copy(mdtv_hbm.at[0, pl.ds(base, BPW)], mdtv_v)

    a = pa_v[...]
    b = pb_v[...]
    t_v[...] = 1.0 / (1.0 + jnp.exp(-jnp.abs(a - b)))
    m_v[...] = jnp.where(b > a,
                         jnp.full((LANES,), 1, jnp.int32),
                         jnp.full((LANES,), 0, jnp.int32))

    init_idx = jnp.full((LANES,), 4, jnp.int32)
    for g in range(NG):
        first_truth = plsc.load_gather(t_v, [init_idx])
        prev = plsc.load_gather(m_v, [init_idx])
        mn = first_truth
        for i in range(L):
            ds = dsym_v[i, pl.ds(sub * BPW + g * LANES, LANES)]
            prev = plsc.load_gather(m_v, [ds * 2 + prev])
            mn = jnp.minimum(mn, plsc.load_gather(t_v, [ds * 2 + prev]))
        mn = jnp.minimum(mn, mdtv_v[pl.ds(g * LANES, LANES)])
        pred_v[pl.ds(g * LANES, LANES)] = mn
        prev_v[pl.ds(g * LANES, LANES)] = prev

    pltpu.sync_copy(pred_v, pred_hbm.at[pl.ds(base, BPW)])
    pltpu.sync_copy(prev_v, prev_hbm.at[pl.ds(base, BPW)])


@functools.cache
def _sc_stage():
    return pl.kernel(
        _sc_body,
        out_type=(jax.ShapeDtypeStruct((B,), jnp.float32),
                  jax.ShapeDtypeStruct((B,), jnp.int32)),
        mesh=plsc.VectorSubcoreMesh(core_axis_name="c", subcore_axis_name="s"),
        compiler_params=pltpu.CompilerParams(needs_layout_passes=False),
        scratch_types=[
            pltpu.VMEM((L, 128), jnp.int32),
            pltpu.VMEM((BPW,), jnp.float32),
            pltpu.VMEM((LANES,), jnp.float32),
            pltpu.VMEM((LANES,), jnp.float32),
            pltpu.VMEM((LANES,), jnp.float32),
            pltpu.VMEM((LANES,), jnp.int32),
            pltpu.VMEM((BPW,), jnp.float32),
            pltpu.VMEM((BPW,), jnp.int32),
        ],
    )


def kernel(binary_list, weight_initial, weights, W_nn, b_nn, eval):
    xt = jnp.transpose(binary_list, (0, 2, 1))
    dsym, mdtv = _tc_stage(xt, W_nn.T, b_nn)
    wflat = weights.reshape(4, 2)
    pad = jnp.zeros((LANES - 5,), jnp.float32)
    pa = jnp.concatenate([wflat[:, 0], weight_initial[:, 0], pad])
    pb = jnp.concatenate([wflat[:, 1], weight_initial[:, 1], pad])
    pred, prev = _sc_stage()(dsym, mdtv, pa, pb)
    return pred, prev.reshape(B, 1, 1)


# speedup vs baseline: 13.8450x; 1.0138x over previous
import functools

import jax
import jax.numpy as jnp
from jax import lax
from jax.experimental import pallas as pl
from jax.experimental.pallas import tpu as pltpu
from jax.experimental.pallas import tpu_sc as plsc

L, B, D = 50, 1024, 784
NC, NS, LANES = 2, 16, 16
NW = NC * NS
BPW = B // NW
NG = BPW // LANES


def _tc_body(x_ref, w_ref, b_ref, dsym_ref, mdtv_ref):
    i = pl.program_id(0)
    x = x_ref[0]
    logits = lax.dot_general(
        w_ref[...], x, (((0,), (0,)), ((), ())),
        preferred_element_type=jnp.float32) + b_ref[...]
    l0 = logits[0]
    l1 = logits[1]
    dtv = jnp.maximum(l0, l1)
    dsym_ref[0, 0, :] = (l1 > l0).astype(jnp.int32)

    @pl.when(i == 0)
    def _():
        mdtv_ref[0, :] = dtv

    @pl.when(i > 0)
    def _():
        mdtv_ref[0, :] = jnp.minimum(mdtv_ref[0, :], dtv)


def _tc_stage(xt, W_nn, b_nn):
    return pl.pallas_call(
        _tc_body,
        grid=(L,),
        in_specs=[
            pl.BlockSpec((1, D, B), lambda i: (i, 0, 0)),
            pl.BlockSpec((D, 2), lambda i: (0, 0)),
            pl.BlockSpec((2, 1), lambda i: (0, 0)),
        ],
        out_specs=[
            pl.BlockSpec((1, 1, B), lambda i: (i, 0, 0)),
            pl.BlockSpec((1, B), lambda i: (0, 0)),
        ],
        out_shape=[
            jax.ShapeDtypeStruct((L, 1, B), jnp.int32),
            jax.ShapeDtypeStruct((1, B), jnp.float32),
        ],
    )(xt, W_nn, b_nn.reshape(2, 1))


def _sc_body(dsym_hbm, mdtv_hbm, w_hbm, wi_hbm, pred_hbm, prev_hbm,
             dsym_v, mdtv_v, wf_v, wi_v, t_v, m_v, pred_v, prev_v):
    wid = lax.axis_index("s") * NC + lax.axis_index("c")
    blk = wid // 4
    sub = wid % 4
    base = blk * 128 + sub * BPW
    pltpu.sync_copy(w_hbm, wf_v)
    pltpu.sync_copy(wi_hbm, wi_v)
    pltpu.sync_copy(
        dsym_hbm.at[:, 0, pl.ds(pl.multiple_of(blk * 128, 128), 128)], dsym_v)
    pltpu.sync_copy(mdtv_hbm.at[0, pl.ds(base, BPW)], mdtv_v)

    ik = lax.iota(jnp.int32, LANES)
    zero16 = jnp.full((LANES,), 0, jnp.int32)
    one16 = jnp.full((LANES,), 1, jnp.int32)
    i0 = jnp.minimum(ik // 2, 1)
    i1 = ik % 2
    aw = plsc.load_gather(wf_v, [i0, i1, zero16])
    bw = plsc.load_gather(wf_v, [i0, i1, one16])
    wi0 = plsc.load_gather(wi_v, [zero16, zero16])
    wi1 = plsc.load_gather(wi_v, [zero16, one16])
    is4 = ik == 4
    a = jnp.where(is4, wi0, aw)
    b = jnp.where(is4, wi1, bw)

    t_v[...] = 1.0 / (1.0 + jnp.exp(-jnp.abs(a - b)))
    m_v[...] = jnp.where(b > a,
                         jnp.full((LANES,), 1, jnp.int32),
                         jnp.full((LANES,), 0, jnp.int32))

    init_idx = jnp.full((LANES,), 4, jnp.int32)
    for g in range(NG):
        first_truth = plsc.load_gather(t_v, [init_idx])
        prev = plsc.load_gather(m_v, [init_idx])
        mn = first_truth
        for i in range(L):
            ds = dsym_v[i, pl.ds(sub * BPW + g * LANES, LANES)]
            prev = plsc.load_gather(m_v, [ds * 2 + prev])
            mn = jnp.minimum(mn, plsc.load_gather(t_v, [ds * 2 + prev]))
        mn = jnp.minimum(mn, mdtv_v[pl.ds(g * LANES, LANES)])
        pred_v[pl.ds(g * LANES, LANES)] = mn
        prev_v[pl.ds(g * LANES, LANES)] = prev

    pltpu.sync_copy(pred_v, pred_hbm.at[pl.ds(base, BPW)])
    pltpu.sync_copy(prev_v, prev_hbm.at[pl.ds(base, BPW)])


@functools.cache
def _sc_stage():
    return pl.kernel(
        _sc_body,
        out_type=(jax.ShapeDtypeStruct((B,), jnp.float32),
                  jax.ShapeDtypeStruct((B,), jnp.int32)),
        mesh=plsc.VectorSubcoreMesh(core_axis_name="c", subcore_axis_name="s"),
        compiler_params=pltpu.CompilerParams(needs_layout_passes=False),
        scratch_types=[
            pltpu.VMEM((L, 128), jnp.int32),
            pltpu.VMEM((BPW,), jnp.float32),
            pltpu.VMEM((2, 2, 2), jnp.float32),
            pltpu.VMEM((1, 2), jnp.float32),
            pltpu.VMEM((LANES,), jnp.float32),
            pltpu.VMEM((LANES,), jnp.int32),
            pltpu.VMEM((BPW,), jnp.float32),
            pltpu.VMEM((BPW,), jnp.int32),
        ],
    )


def kernel(binary_list, weight_initial, weights, W_nn, b_nn, eval):
    xt = jnp.transpose(binary_list, (0, 2, 1))
    dsym, mdtv = _tc_stage(xt, W_nn, b_nn)
    pred, prev = _sc_stage()(dsym, mdtv, weights, weight_initial)
    return pred, prev.reshape(B, 1, 1)
